# Initial kernel scaffold; baseline (speedup 1.0000x reference)
import jax
import jax.numpy as jnp
from jax.experimental import pallas as pl


def kernel(x, edge_index, edge_attr, switch_action, train_action, Wm1, bm1, Ws1, bs1, Wm2, bm2, Ws2, bs2, W1, b1, W2, b2, W3, b3):
    raise NotImplementedError("write your pallas kernel here")


# speedup vs baseline: 9.0652x; 9.0652x over previous
import jax
import jax.numpy as jnp
from jax import lax
from jax.experimental import pallas as pl
from jax.experimental.pallas import tpu as pltpu
from jax.experimental.pallas import tpu_sc as plsc

N = 10000
E = 320000
NP = 10240
NC, NS = 2, 16
NWK = NC * NS
EPW = E // NWK
W = 80
R = 5
WPW = EPW // W
GPW = WPW // R
GE = R * W
F32 = jnp.float32

_mesh = plsc.VectorSubcoreMesh(core_axis_name="c", subcore_axis_name="s")


def _make_sc_scatter(with_ea: bool):
    out_type = [jax.ShapeDtypeStruct((NC, NP, 64), F32)]
    scratch = [
        pltpu.VMEM((WPW, W), jnp.int32),
        pltpu.VMEM((WPW, W), jnp.int32),
        [pltpu.VMEM((W, 64), F32) for _ in range(R)],
        [pltpu.SemaphoreType.DMA for _ in range(R)],
        pltpu.SemaphoreType.DMA,
        pltpu.VMEM_SHARED((NP, 64), F32),
    ]
    if with_ea:
        out_type += [jax.ShapeDtypeStruct((NC, NP, 16), F32),
                     jax.ShapeDtypeStruct((NC, NP), F32)]
        scratch += [
            pltpu.VMEM((GE, 16), F32),
            pltpu.SemaphoreType.DMA,
            pltpu.VMEM((W,), F32),
            pltpu.VMEM_SHARED((NP, 16), F32),
            pltpu.VMEM_SHARED((NP,), F32),
        ]

    def body(src3d, dst3d, y_hbm, z64, z16, z1, ea2, agg_out, ea_out,
             deg_out, src_all, dst_all, rows, gsems, ssem, agg_sh,
             eab=None, esem=None, ones_v=None, ea_sh=None, deg_sh=None):
        c = lax.axis_index("c")
        s = lax.axis_index("s")
        wid = s * NC + c

        zr = NP // NS
        pltpu.sync_copy(z64.at[pl.ds(s * zr, zr), :],
                        agg_sh.at[pl.ds(s * zr, zr), :])
        if with_ea:
            pltpu.sync_copy(z16.at[pl.ds(s * zr, zr), :],
                            ea_sh.at[pl.ds(s * zr, zr), :])
            pltpu.sync_copy(z1.at[pl.ds(s * zr, zr)],
                            deg_sh.at[pl.ds(s * zr, zr)])
            for i in range(W // 16):
                ones_v[pl.ds(i * 16, 16)] = jnp.ones((16,), F32)
        pltpu.sync_copy(src3d.at[wid], src_all)
        pltpu.sync_copy(dst3d.at[wid], dst_all)
        plsc.subcore_barrier()

        def group(g, carry):
            hs = []
            if with_ea:
                h_e = pltpu.async_copy(
                    ea2.at[pl.ds((wid * GPW + g) * GE, GE), :], eab, esem)
            h_g = [pltpu.async_copy(y_hbm.at[src_all.at[g * R + j]],
                                    rows[j], gsems[j]) for j in range(R)]
            if with_ea:
                h_e.wait()
                for j in range(R):
                    dref = dst_all.at[g * R + j]
                    hs.append(pltpu.async_copy(eab.at[pl.ds(j * W, W), :],
                                               ea_sh.at[dref], ssem,
                                               add=True))
                    hs.append(pltpu.async_copy(ones_v, deg_sh.at[dref],
                                               ssem, add=True))
            for j in range(R):
                h_g[j].wait()
                hs.append(pltpu.async_copy(rows[j],
                                           agg_sh.at[dst_all.at[g * R + j]],
                                           ssem, add=True))
            for h in hs:
                h.wait()
            return carry

        lax.fori_loop(0, GPW, group, 0)
        plsc.subcore_barrier()

        pltpu.sync_copy(agg_sh.at[pl.ds(s * zr, zr), :],
                        agg_out.at[c, pl.ds(s * zr, zr), :])
        if with_ea:
            pltpu.sync_copy(ea_sh.at[pl.ds(s * zr, zr), :],
                            ea_out.at[c, pl.ds(s * zr, zr), :])
            pltpu.sync_copy(deg_sh.at[pl.ds(s * zr, zr)],
                            deg_out.at[c, pl.ds(s * zr, zr)])

    if with_ea:
        def body_ea(src3d, dst3d, y_hbm, z64, z16, z1, ea2, agg_out, ea_out,
                    deg_out, src_all, dst_all, rows, gsems, ssem, agg_sh,
                    eab, esem, ones_v, ea_sh, deg_sh):
            body(src3d, dst3d, y_hbm, z64, z16, z1, ea2, agg_out, ea_out,
                 deg_out, src_all, dst_all, rows, gsems, ssem, agg_sh,
                 eab, esem, ones_v, ea_sh, deg_sh)
        fn = body_ea
    else:
        def body_noea(src3d, dst3d, y_hbm, z64, agg_out, src_all, dst_all,
                      rows, gsems, ssem, agg_sh):
            body(src3d, dst3d, y_hbm, z64, None, None, None, agg_out, None,
                 None, src_all, dst_all, rows, gsems, ssem, agg_sh)
        fn = body_noea

    return pl.kernel(fn, out_type=tuple(out_type) if with_ea else out_type[0],
                     mesh=_mesh, scratch_types=scratch,
                     compiler_params=pltpu.CompilerParams(
                         use_tc_tiling_on_sc=False))


_sc1 = _make_sc_scatter(True)
_sc2 = _make_sc_scatter(False)


def _tc1_body(x_ref, wmx_ref, ws_ref, y_ref, s_ref):
    x = x_ref[...]
    y_ref[...] = jnp.dot(x, wmx_ref[...], preferred_element_type=F32)
    s_ref[...] = jnp.dot(x, ws_ref[...], preferred_element_type=F32)


_tc1 = pl.pallas_call(
    _tc1_body,
    out_shape=(jax.ShapeDtypeStruct((N, 64), F32),
               jax.ShapeDtypeStruct((N, 64), F32)),
)


def _tc2_body(s1_ref, aggp_ref, eap_ref, degp_ref, wme_ref, bm_ref, bs_ref,
              wmx2_ref, ws2_ref, y2_ref, s2_ref):
    agg = aggp_ref[0, :N] + aggp_ref[1, :N]
    ea = eap_ref[0, :N] + eap_ref[1, :N]
    deg = degp_ref[0, :N] + degp_ref[1, :N]
    extra = (jnp.dot(ea, wme_ref[...], preferred_element_type=F32)
             + deg[:, None] * bm_ref[...])
    h = jnp.maximum(s1_ref[...] + bs_ref[...] + agg + extra, 0.0)
    y2_ref[...] = jnp.dot(h, wmx2_ref[...], preferred_element_type=F32)
    s2_ref[...] = jnp.dot(h, ws2_ref[...], preferred_element_type=F32)


_tc2 = pl.pallas_call(
    _tc2_body,
    out_shape=(jax.ShapeDtypeStruct((N, 64), F32),
               jax.ShapeDtypeStruct((N, 64), F32)),
)


def _tc3_body(s2_ref, aggp_ref, eap_ref, degp_ref, wme_ref, bm_ref, bs_ref,
              w1g_ref, w1s_ref, w1t_ref, b1_ref, w2_ref, b2_ref, w3_ref,
              b3_ref, sa_ref, ta_ref, out_ref):
    agg = aggp_ref[0, :N] + aggp_ref[1, :N]
    ea = eap_ref[0, :N] + eap_ref[1, :N]
    deg = degp_ref[0, :N] + degp_ref[1, :N]
    extra = (jnp.dot(ea, wme_ref[...], preferred_element_type=F32)
             + deg[:, None] * bm_ref[...])
    h = jnp.maximum(s2_ref[...] + bs_ref[...] + agg + extra, 0.0)
    g = jnp.mean(h, axis=0, keepdims=True)
    v = jnp.dot(g, w1g_ref[...], preferred_element_type=F32)
    v = v + jnp.dot(sa_ref[...], w1s_ref[...], preferred_element_type=F32)
    v = v + jnp.dot(ta_ref[...], w1t_ref[...], preferred_element_type=F32)
    v = jnp.maximum(v + b1_ref[...], 0.0)
    v = jnp.maximum(jnp.dot(v, w2_ref[...], preferred_element_type=F32)
                    + b2_ref[...], 0.0)
    out_ref[...] = (jnp.dot(v, w3_ref[...], preferred_element_type=F32)
                    + b3_ref[...])


_tc3 = pl.pallas_call(
    _tc3_body,
    out_shape=jax.ShapeDtypeStruct((1, 1), F32),
)


def kernel(x, edge_index, edge_attr, switch_action, train_action,
           Wm1, bm1, Ws1, bs1, Wm2, bm2, Ws2, bs2,
           W1, b1, W2, b2, W3, b3):
    src3d = edge_index[0].reshape(NWK, WPW, W)
    dst3d = edge_index[1].reshape(NWK, WPW, W)
    z64 = jnp.zeros((NP, 64), F32)
    z16 = jnp.zeros((NP, 16), F32)
    z1 = jnp.zeros((NP,), F32)

    y1, s1 = _tc1(x, Wm1[:128], Ws1)
    aggp1, eap, degp = _sc1(src3d, dst3d, y1, z64, z16, z1, edge_attr)
    y2, s2 = _tc2(s1, aggp1, eap, degp, Wm1[128:], bm1.reshape(1, 64),
                  bs1.reshape(1, 64), Wm2[:64], Ws2)
    aggp2 = _sc2(src3d, dst3d, y2, z64)
    out = _tc3(s2, aggp2, eap, degp, Wm2[64:], bm2.reshape(1, 64),
               bs2.reshape(1, 64), W1[:64], W1[64:96], W1[96:],
               b1.reshape(1, 128), W2, b2.reshape(1, 64), W3,
               b3.reshape(1, 1), switch_action.reshape(1, 32),
               train_action.reshape(1, 16))
    return out.reshape(-1)


# speedup vs baseline: 9.8491x; 1.0865x over previous
import jax
import jax.numpy as jnp
from jax import lax
from jax.experimental import pallas as pl
from jax.experimental.pallas import tpu as pltpu
from jax.experimental.pallas import tpu_sc as plsc

N = 10000
E = 320000
NP = 10240
NC, NS = 2, 16
NWK = NC * NS
EPW = E // NWK
W = 80
R = 5
WPW = EPW // W
GPW = WPW // R
GE = R * W
F32 = jnp.float32

_mesh = plsc.VectorSubcoreMesh(core_axis_name="c", subcore_axis_name="s")


def _make_sc_scatter(with_ea: bool):
    out_type = [pltpu.HBM((NC, NP, 64), F32)]
    scratch = [
        [pltpu.VMEM((R, W), jnp.int32) for _ in range(2)],
        [pltpu.VMEM((R, W), jnp.int32) for _ in range(2)],
        [pltpu.SemaphoreType.DMA for _ in range(2)],
        [[pltpu.VMEM((W, 64), F32) for _ in range(R)] for _ in range(2)],
        [[pltpu.SemaphoreType.DMA for _ in range(R)] for _ in range(2)],
        [pltpu.SemaphoreType.DMA for _ in range(2)],
        pltpu.VMEM_SHARED((NP, 64), F32),
    ]
    if with_ea:
        out_type += [pltpu.HBM((NC, NP, 16), F32),
                     pltpu.HBM((NC, NP), F32)]
        scratch += [
            [pltpu.VMEM((GE, 16), F32) for _ in range(2)],
            [pltpu.SemaphoreType.DMA for _ in range(2)],
            pltpu.VMEM((W,), F32),
            pltpu.VMEM_SHARED((NP, 16), F32),
            pltpu.VMEM_SHARED((NP,), F32),
        ]

    def body(src4d, dst4d, y_hbm, z64, z16, z1, ea2, agg_out, ea_out,
             deg_out, srcg, dstg, isem, rows, gsems, ssem, agg_sh,
             eab=None, esem=None, ones_v=None, ea_sh=None, deg_sh=None):
        c = lax.axis_index("c")
        s = lax.axis_index("s")
        wid = s * NC + c

        zr = NP // NS
        pltpu.sync_copy(z64.at[pl.ds(s * zr, zr), :],
                        agg_sh.at[pl.ds(s * zr, zr), :])
        if with_ea:
            pltpu.sync_copy(z16.at[pl.ds(s * zr, zr), :],
                            ea_sh.at[pl.ds(s * zr, zr), :])
            pltpu.sync_copy(z1.at[pl.ds(s * zr, zr)],
                            deg_sh.at[pl.ds(s * zr, zr)])
            for i in range(W // 16):
                ones_v[pl.ds(i * 16, 16)] = jnp.ones((16,), F32)
        plsc.subcore_barrier()

        def _ea_src(g):
            return ea2.at[pl.ds((wid * GPW + g) * GE, GE), :]

        def load(g, sl):
            pltpu.async_copy(src4d.at[wid, g], srcg[sl], isem[sl])
            pltpu.async_copy(dst4d.at[wid, g], dstg[sl], isem[sl])
            if with_ea:
                pltpu.async_copy(_ea_src(g), eab[sl], esem[sl])

        def gather(g, sl):
            pltpu.make_async_copy(src4d.at[wid, g], srcg[sl], isem[sl]).wait()
            pltpu.make_async_copy(dst4d.at[wid, g], dstg[sl], isem[sl]).wait()
            for j in range(R):
                pltpu.async_copy(y_hbm.at[srcg[sl].at[j]],
                                 rows[sl][j], gsems[sl][j])

        def scatter(g, sl):
            if with_ea:
                pltpu.make_async_copy(_ea_src(g), eab[sl], esem[sl]).wait()
                for j in range(R):
                    dref = dstg[sl].at[j]
                    pltpu.async_copy(eab[sl].at[pl.ds(j * W, W), :],
                                     ea_sh.at[dref], ssem[sl], add=True)
                    pltpu.async_copy(ones_v, deg_sh.at[dref], ssem[sl],
                                     add=True)
            for j in range(R):
                pltpu.make_async_copy(y_hbm.at[srcg[sl].at[j]],
                                      rows[sl][j], gsems[sl][j]).wait()
                pltpu.async_copy(rows[sl][j], agg_sh.at[dstg[sl].at[j]],
                                 ssem[sl], add=True)

        def drain(sl):
            for j in range(R):
                dref = dstg[sl].at[j]
                if with_ea:
                    pltpu.make_async_copy(eab[sl].at[pl.ds(j * W, W), :],
                                          ea_sh.at[dref], ssem[sl]).wait()
                    pltpu.make_async_copy(ones_v, deg_sh.at[dref],
                                          ssem[sl]).wait()
                pltpu.make_async_copy(rows[sl][j], agg_sh.at[dref],
                                      ssem[sl]).wait()

        load(0, 0)
        gather(0, 0)
        load(1, 1)

        def pair(i, carry):
            g0 = 2 * i
            gather(g0 + 1, 1)
            scatter(g0, 0)
            drain(0)
            load(g0 + 2, 0)
            scatter(g0 + 1, 1)
            drain(1)
            load(g0 + 3, 1)
            gather(g0 + 2, 0)
            return carry

        lax.fori_loop(0, (GPW - 3) // 2, pair, 0)
        gather(GPW - 2, 1)
        scatter(GPW - 3, 0)
        drain(0)
        load(GPW - 1, 0)
        scatter(GPW - 2, 1)
        drain(1)
        gather(GPW - 1, 0)
        scatter(GPW - 1, 0)
        drain(0)
        plsc.subcore_barrier()

        pltpu.sync_copy(agg_sh.at[pl.ds(s * zr, zr), :],
                        agg_out.at[c, pl.ds(s * zr, zr), :])
        if with_ea:
            pltpu.sync_copy(ea_sh.at[pl.ds(s * zr, zr), :],
                            ea_out.at[c, pl.ds(s * zr, zr), :])
            pltpu.sync_copy(deg_sh.at[pl.ds(s * zr, zr)],
                            deg_out.at[c, pl.ds(s * zr, zr)])

    if with_ea:
        def body_ea(src4d, dst4d, y_hbm, z64, z16, z1, ea2, agg_out, ea_out,
                    deg_out, srcg, dstg, isem, rows, gsems, ssem, agg_sh,
                    eab, esem, ones_v, ea_sh, deg_sh):
            body(src4d, dst4d, y_hbm, z64, z16, z1, ea2, agg_out, ea_out,
                 deg_out, srcg, dstg, isem, rows, gsems, ssem, agg_sh,
                 eab, esem, ones_v, ea_sh, deg_sh)
        fn = body_ea
    else:
        def body_noea(src4d, dst4d, y_hbm, z64, agg_out, srcg, dstg, isem,
                      rows, gsems, ssem, agg_sh):
            body(src4d, dst4d, y_hbm, z64, None, None, None, agg_out, None,
                 None, srcg, dstg, isem, rows, gsems, ssem, agg_sh)
        fn = body_noea

    return pl.kernel(fn, out_type=tuple(out_type) if with_ea else out_type[0],
                     mesh=_mesh, scratch_types=scratch,
                     compiler_params=pltpu.CompilerParams(
                         use_tc_tiling_on_sc=False))


_sc1 = _make_sc_scatter(True)
_sc2 = _make_sc_scatter(False)


def _tc1_body(x_ref, wmx_ref, ws_ref, y_ref, s_ref):
    x = x_ref[...]
    y_ref[...] = jnp.dot(x, wmx_ref[...], preferred_element_type=F32)
    s_ref[...] = jnp.dot(x, ws_ref[...], preferred_element_type=F32)


_tc1 = pl.pallas_call(
    _tc1_body,
    out_shape=(jax.ShapeDtypeStruct((N, 64), F32),
               jax.ShapeDtypeStruct((N, 64), F32)),
)


def _tc2_body(s1_ref, aggp_ref, eap_ref, degp_ref, wme_ref, bm_ref, bs_ref,
              wmx2_ref, ws2_ref, y2_ref, s2_ref):
    agg = aggp_ref[0, :N] + aggp_ref[1, :N]
    ea = eap_ref[0, :N] + eap_ref[1, :N]
    deg = degp_ref[0, :N] + degp_ref[1, :N]
    extra = (jnp.dot(ea, wme_ref[...], preferred_element_type=F32)
             + deg[:, None] * bm_ref[...])
    h = jnp.maximum(s1_ref[...] + bs_ref[...] + agg + extra, 0.0)
    y2_ref[...] = jnp.dot(h, wmx2_ref[...], preferred_element_type=F32)
    s2_ref[...] = jnp.dot(h, ws2_ref[...], preferred_element_type=F32)


_tc2 = pl.pallas_call(
    _tc2_body,
    out_shape=(jax.ShapeDtypeStruct((N, 64), F32),
               jax.ShapeDtypeStruct((N, 64), F32)),
)


def _tc3_body(s2_ref, aggp_ref, eap_ref, degp_ref, wme_ref, bm_ref, bs_ref,
              w1g_ref, w1s_ref, w1t_ref, b1_ref, w2_ref, b2_ref, w3_ref,
              b3_ref, sa_ref, ta_ref, out_ref):
    agg = aggp_ref[0, :N] + aggp_ref[1, :N]
    ea = eap_ref[0, :N] + eap_ref[1, :N]
    deg = degp_ref[0, :N] + degp_ref[1, :N]
    extra = (jnp.dot(ea, wme_ref[...], preferred_element_type=F32)
             + deg[:, None] * bm_ref[...])
    h = jnp.maximum(s2_ref[...] + bs_ref[...] + agg + extra, 0.0)
    g = jnp.mean(h, axis=0, keepdims=True)
    v = jnp.dot(g, w1g_ref[...], preferred_element_type=F32)
    v = v + jnp.dot(sa_ref[...], w1s_ref[...], preferred_element_type=F32)
    v = v + jnp.dot(ta_ref[...], w1t_ref[...], preferred_element_type=F32)
    v = jnp.maximum(v + b1_ref[...], 0.0)
    v = jnp.maximum(jnp.dot(v, w2_ref[...], preferred_element_type=F32)
                    + b2_ref[...], 0.0)
    out_ref[...] = (jnp.dot(v, w3_ref[...], preferred_element_type=F32)
                    + b3_ref[...])


_tc3 = pl.pallas_call(
    _tc3_body,
    out_shape=jax.ShapeDtypeStruct((1, 1), F32),
)


def kernel(x, edge_index, edge_attr, switch_action, train_action,
           Wm1, bm1, Ws1, bs1, Wm2, bm2, Ws2, bs2,
           W1, b1, W2, b2, W3, b3):
    src4d = edge_index[0].reshape(NWK, GPW, R, W)
    dst4d = edge_index[1].reshape(NWK, GPW, R, W)
    z64 = jnp.zeros((NP, 64), F32)
    z16 = jnp.zeros((NP, 16), F32)
    z1 = jnp.zeros((NP,), F32)

    y1, s1 = _tc1(x, Wm1[:128], Ws1)
    aggp1, eap, degp = _sc1(src4d, dst4d, y1, z64, z16, z1, edge_attr)
    y2, s2 = _tc2(s1, aggp1, eap, degp, Wm1[128:], bm1.reshape(1, 64),
                  bs1.reshape(1, 64), Wm2[:64], Ws2)
    aggp2 = _sc2(src4d, dst4d, y2, z64)
    out = _tc3(s2, aggp2, eap, degp, Wm2[64:], bm2.reshape(1, 64),
               bs2.reshape(1, 64), W1[:64], W1[64:96], W1[96:],
               b1.reshape(1, 128), W2, b2.reshape(1, 64), W3,
               b3.reshape(1, 1), switch_action.reshape(1, 32),
               train_action.reshape(1, 16))
    return out.reshape(-1)
